# baseline (speedup 1.0000x reference)
import jax
import jax.numpy as jnp
from jax.experimental import pallas as pl
from jax.experimental.pallas import tpu as pltpu

_BETA = 1.0 / 9.0


def _huber_kernel(cw_sref, x_ref, t_ref, w_ref, o_ref, *, beta, nc):
    w = w_ref[...].astype(jnp.float32)
    for c in range(nc):
        x = x_ref[c].astype(jnp.float32)
        t = t_ref[c].astype(jnp.float32)
        t = jnp.where(t != t, x, t)
        n = jnp.abs((x - t) * cw_sref[c])
        loss = jnp.where(n < beta, (0.5 / beta) * n * n, n - 0.5 * beta)
        o_ref[c] = (loss * w).astype(o_ref.dtype)


def _anchor_tile(a, c_planes, itemsize=4):
    budget = 12 * 1024 * 1024
    best = None
    for la in range(128, a + 1, 128):
        if a % la:
            continue
        if (a // la) % 2:
            continue
        need = 2 * la * 16 * itemsize * (3 * c_planes + 1)
        if need > budget:
            continue
        if best is None or la > best:
            best = la
    if best is not None:
        return best
    return 128


def kernel(inp, target, weights, code_weights):
    B, A, C = inp.shape
    assert A % 256 == 0, "unsupported geometry"
    la = _anchor_tile(A, C)
    ka = A // la
    out_dtype = inp.dtype

    x3 = jnp.transpose(inp, (2, 0, 1))
    t3 = jnp.transpose(target, (2, 0, 1))
    cw = code_weights.astype(jnp.float32)

    body = lambda s, x, t, w, o: _huber_kernel(s, x, t, w, o,
                                               beta=float(_BETA), nc=C)
    out3 = pl.pallas_call(
        body,
        out_shape=jax.ShapeDtypeStruct((C, B, A), out_dtype),
        grid_spec=pltpu.PrefetchScalarGridSpec(
            num_scalar_prefetch=1,
            grid=(ka,),
            in_specs=[
                pl.BlockSpec((C, B, la), lambda a, *_: (0, 0, a)),
                pl.BlockSpec((C, B, la), lambda a, *_: (0, 0, a)),
                pl.BlockSpec((B, la), lambda a, *_: (0, a)),
            ],
            out_specs=pl.BlockSpec((C, B, la), lambda a, *_: (0, 0, a)),
        ),
        compiler_params=pltpu.CompilerParams(
            dimension_semantics=("parallel",),
            vmem_limit_bytes=50 * 1024 * 1024),
    )(cw, x3, t3, weights)

    return jnp.transpose(out3, (1, 2, 0))


# speedup vs baseline: 1.3935x; 1.3935x over previous
import jax
import jax.numpy as jnp
from jax.experimental import pallas as pl
from jax.experimental.pallas import tpu as pltpu

_BETA = 1.0 / 9.0


def _huber_kernel(cw_sref, x_ref, t_ref, w_ref, o_ref, *, beta, nc):
    w = w_ref[...].astype(jnp.float32)
    for c in range(nc):
        x = x_ref[c].astype(jnp.float32)
        t = t_ref[c].astype(jnp.float32)
        t = jnp.where(t != t, x, t)
        n = jnp.abs((x - t) * cw_sref[c])
        loss = jnp.where(n < beta, (0.5 / beta) * n * n, n - 0.5 * beta)
        o_ref[c] = (loss * w).astype(o_ref.dtype)


def _anchor_tile(a, c_planes, itemsize=4):
    budget = 50 * 1024 * 1024
    best = None
    for la in range(128, a + 1, 128):
        blocks = -(-a // la)
        if blocks % 2:
            continue
        need = 2 * la * 16 * itemsize * (3 * c_planes + 1)
        if need > budget:
            continue
        if best is None or la > best:
            best = la
    if best is not None:
        return best
    return 128


def kernel(inp, target, weights, code_weights):
    B, A, C = inp.shape
    assert A % 256 == 0, "unsupported geometry"
    la = _anchor_tile(A, C)
    ka = A // la
    out_dtype = inp.dtype

    x3 = jnp.transpose(inp, (2, 0, 1))
    t3 = jnp.transpose(target, (2, 0, 1))
    cw = code_weights.astype(jnp.float32)

    body = lambda s, x, t, w, o: _huber_kernel(s, x, t, w, o,
                                               beta=float(_BETA), nc=C)
    out3 = pl.pallas_call(
        body,
        out_shape=jax.ShapeDtypeStruct((C, B, A), out_dtype),
        grid_spec=pltpu.PrefetchScalarGridSpec(
            num_scalar_prefetch=1,
            grid=(ka,),
            in_specs=[
                pl.BlockSpec((C, B, la), lambda a, *_: (0, 0, a)),
                pl.BlockSpec((C, B, la), lambda a, *_: (0, 0, a)),
                pl.BlockSpec((B, la), lambda a, *_: (0, a)),
            ],
            out_specs=pl.BlockSpec((C, B, la), lambda a, *_: (0, 0, a)),
        ),
        compiler_params=pltpu.CompilerParams(
            dimension_semantics=("parallel",),
            vmem_limit_bytes=57 * 1024 * 1024),
    )(cw, x3, t3, weights)

    return jnp.transpose(out3, (1, 2, 0))
